# baseline (speedup 1.0000x reference)
import functools

import jax
import jax.numpy as jnp
from jax import lax
from jax.experimental import pallas as pl
from jax.experimental.pallas import tpu as pltpu
from jax.experimental.pallas import tpu_sc as plsc

_B, _T, _C = 2, 2048, 1024
_E = 8
_K = 2
_DFF = 4 * _C
_N = _B * _T
_CAP = int(_T / _E * 1.25) * _B
_CPAD = _CAP + 8
_NROWS = _E * _CPAD

_NC, _NS = 2, 16
_NW = _NC * _NS


_RT_BM, _RT_BN = 512, 256


def _router_kernel(x_ref, w1_ref, b1_ref, w2_ref, b2_ref, w3_ref, b3_ref,
                   o_ref, h1_s):
    n = pl.program_id(1)

    @pl.when(n == 0)
    def _():
        a = jnp.dot(x_ref[...], w1_ref[...], preferred_element_type=jnp.float32)
        h1_s[...] = jnp.maximum(a + b1_ref[...], 0.0)

    h2 = jnp.dot(h1_s[...], w2_ref[...], preferred_element_type=jnp.float32)
    b2 = b2_ref[0, :, pl.ds(n * _RT_BN, _RT_BN)]
    h2 = jnp.maximum(h2 + b2, 0.0)
    part = jnp.dot(h2, w3_ref[...], preferred_element_type=jnp.float32)

    @pl.when(n == 0)
    def _():
        o_ref[...] = part + b3_ref[...]

    @pl.when(n != 0)
    def _():
        o_ref[...] += part


def _router_logits(xt, Wr1, br1, Wr2, br2, Wr3, br3):
    BM, BN = _RT_BM, _RT_BN
    return pl.pallas_call(
        _router_kernel,
        grid=(_N // BM, _DFF // BN),
        in_specs=[
            pl.BlockSpec((BM, _C), lambda m, n: (m, 0)),
            pl.BlockSpec((_C, _DFF), lambda m, n: (0, 0)),
            pl.BlockSpec((1, _DFF), lambda m, n: (0, 0)),
            pl.BlockSpec((_DFF, BN), lambda m, n: (0, n)),
            pl.BlockSpec((1, 1, _DFF), lambda m, n: (0, 0, 0)),
            pl.BlockSpec((BN, _E), lambda m, n: (n, 0)),
            pl.BlockSpec((1, _E), lambda m, n: (0, 0)),
        ],
        out_specs=pl.BlockSpec((BM, _E), lambda m, n: (m, 0)),
        out_shape=jax.ShapeDtypeStruct((_N, _E), jnp.float32),
        scratch_shapes=[pltpu.VMEM((BM, _DFF), jnp.float32)],
    )(xt, Wr1, br1.reshape(1, _DFF), Wr2, br2.reshape(1, 1, _DFF),
      Wr3, br3.reshape(1, _E))


def _routing_kernel(lg_ref, d1_ref, d2_ref, g1_ref, g2_ref):
    lg = lg_ref[...]
    m = jnp.max(lg, axis=1, keepdims=True)
    p = jnp.exp(lg - m)
    probs = p / jnp.sum(p, axis=1, keepdims=True)
    eidx = lax.broadcasted_iota(jnp.int32, (_N, _E), 1)

    p1 = jnp.max(probs, axis=1, keepdims=True)
    e1 = jnp.min(jnp.where(probs == p1, eidx, _E), axis=1, keepdims=True)
    oh1 = eidx == e1
    probs_m = jnp.where(oh1, -1.0, probs)
    p2 = jnp.max(probs_m, axis=1, keepdims=True)
    e2 = jnp.min(jnp.where(probs_m == p2, eidx, _E), axis=1, keepdims=True)
    oh2 = eidx == e2

    cnt = oh1.astype(jnp.float32) + oh2.astype(jnp.float32)
    c = cnt
    sh = 1
    while sh < _N:
        c = c + jnp.concatenate(
            [jnp.zeros((sh, _E), jnp.float32), c[: _N - sh, :]], axis=0)
        sh *= 2
    cex = c - cnt

    pos1 = jnp.sum(jnp.where(oh1, cex, 0.0), axis=1, keepdims=True)
    pos2 = jnp.sum(jnp.where(oh2, cex, 0.0), axis=1, keepdims=True)
    g1 = jnp.sum(jnp.where(oh1, probs, 0.0), axis=1, keepdims=True)
    g2 = jnp.sum(jnp.where(oh2, probs, 0.0), axis=1, keepdims=True)

    keep1 = pos1 < float(_CAP)
    keep2 = pos2 < float(_CAP)
    p1i = pos1.astype(jnp.int32)
    p2i = pos2.astype(jnp.int32)
    d1 = e1 * _CPAD + jnp.where(keep1, p1i, _CAP)
    d2 = e2 * _CPAD + jnp.where(keep2, p2i, _CAP)

    d1_ref[...] = d1.reshape(_N)
    d2_ref[...] = d2.reshape(_N)
    g1_ref[...] = jnp.broadcast_to(jnp.where(keep1, g1, 0.0), (_N, 16))
    g2_ref[...] = jnp.broadcast_to(jnp.where(keep2, g2, 0.0), (_N, 16))


def _routing(logits):
    return pl.pallas_call(
        _routing_kernel,
        out_shape=(
            jax.ShapeDtypeStruct((_N,), jnp.int32),
            jax.ShapeDtypeStruct((_N,), jnp.int32),
            jax.ShapeDtypeStruct((_N, 16), jnp.float32),
            jax.ShapeDtypeStruct((_N, 16), jnp.float32),
        ),
    )(logits)


def _dispatch(tokens, d1, d2):
    W = _C // 2
    mesh = plsc.VectorSubcoreMesh(core_axis_name="c", subcore_axis_name="s")
    CH = 32
    tpw = _N // _NW
    NCH = tpw // CH

    @functools.partial(
        pl.kernel,
        out_type=jax.ShapeDtypeStruct((_NROWS, W), jnp.int32),
        mesh=mesh,
        scratch_types=[
            pltpu.VMEM((CH, W), jnp.int32),
            pltpu.VMEM((CH, W), jnp.int32),
            [pltpu.VMEM((CH,), jnp.int32) for _ in range(NCH)],
            [pltpu.VMEM((CH,), jnp.int32) for _ in range(NCH)],
            pltpu.SemaphoreType.DMA,
            pltpu.SemaphoreType.DMA,
            pltpu.SemaphoreType.DMA,
            pltpu.SemaphoreType.DMA,
        ],
    )
    def k(tok_hbm, d1_hbm, d2_hbm, buf_hbm, rA, rB, i1s, i2s, sL0, sL1, sS, sI):
        wid = lax.axis_index("s") * _NC + lax.axis_index("c")
        base0 = wid * tpw
        rows = [rA, rB]
        sL = [sL0, sL1]
        ih = []
        for c in range(NCH):
            ih.append(pltpu.async_copy(d1_hbm.at[pl.ds(base0 + c * CH, CH)], i1s[c], sI))
            ih.append(pltpu.async_copy(d2_hbm.at[pl.ds(base0 + c * CH, CH)], i2s[c], sI))
        for h in ih:
            h.wait()
        load = [None] * NCH
        load[0] = pltpu.async_copy(tok_hbm.at[pl.ds(base0, CH)], rows[0], sL[0])
        scat = []
        for c in range(NCH):
            load[c].wait()
            if c >= 1:
                scat[2 * (c - 1)].wait()
                scat[2 * (c - 1) + 1].wait()
            if c + 1 < NCH:
                load[c + 1] = pltpu.async_copy(
                    tok_hbm.at[pl.ds(base0 + (c + 1) * CH, CH)],
                    rows[(c + 1) % 2], sL[(c + 1) % 2])
            scat.append(pltpu.async_copy(rows[c % 2], buf_hbm.at[i1s[c]], sS))
            scat.append(pltpu.async_copy(rows[c % 2], buf_hbm.at[i2s[c]], sS))
        scat[-2].wait()
        scat[-1].wait()

    return k(tokens, d1, d2)


_FFN_BN = 2048


def _ffn_kernel(buf_ref, w1_ref, b1_ref, w2_ref, b2_ref, o_ref):
    n = pl.program_id(1)
    w1 = w1_ref[0].astype(jnp.bfloat16)
    h = jnp.dot(buf_ref[...], w1, preferred_element_type=jnp.float32)
    b1 = b1_ref[0, :, pl.ds(n * _FFN_BN, _FFN_BN)]
    h = jnp.maximum(h + b1, 0.0).astype(jnp.bfloat16)
    w2 = w2_ref[0].astype(jnp.bfloat16)
    part = jnp.dot(h, w2, preferred_element_type=jnp.float32)

    @pl.when(n == 0)
    def _():
        o_ref[...] = part + b2_ref[0]

    @pl.when(n != 0)
    def _():
        o_ref[...] += part


def _ffn(buf, We1, be1, We2, be2):
    BN = _FFN_BN
    return pl.pallas_call(
        _ffn_kernel,
        grid=(_E, _DFF // BN),
        in_specs=[
            pl.BlockSpec((_CPAD, _C), lambda e, n: (e, 0)),
            pl.BlockSpec((1, _C, BN), lambda e, n: (e, 0, n)),
            pl.BlockSpec((1, 1, _DFF), lambda e, n: (e, 0, 0)),
            pl.BlockSpec((1, BN, _C), lambda e, n: (e, n, 0)),
            pl.BlockSpec((1, 1, _C), lambda e, n: (e, 0, 0)),
        ],
        out_specs=pl.BlockSpec((_CPAD, _C), lambda e, n: (e, 0)),
        out_shape=jax.ShapeDtypeStruct((_NROWS, _C), jnp.float32),
    )(buf, We1.reshape(_E, _C, _DFF), be1.reshape(_E, 1, _DFF),
      We2.reshape(_E, _DFF, _C), be2.reshape(_E, 1, _C))


def _combine(eo, d1, d2, g1, g2):
    mesh = plsc.VectorSubcoreMesh(core_axis_name="c", subcore_axis_name="s")
    CH = 16
    tpw = _N // _NW
    NCH = tpw // CH

    @functools.partial(
        pl.kernel,
        out_type=jax.ShapeDtypeStruct((_N, _C), jnp.float32),
        mesh=mesh,
        scratch_types=[
            [pltpu.VMEM((CH, _C), jnp.float32) for _ in range(2)],
            [pltpu.VMEM((CH, _C), jnp.float32) for _ in range(2)],
            pltpu.VMEM((CH, _C), jnp.float32),
            [pltpu.VMEM((CH,), jnp.int32) for _ in range(NCH)],
            [pltpu.VMEM((CH,), jnp.int32) for _ in range(NCH)],
            pltpu.VMEM((tpw, 16), jnp.float32),
            pltpu.VMEM((tpw, 16), jnp.float32),
            [pltpu.SemaphoreType.DMA for _ in range(2)],
            [pltpu.SemaphoreType.DMA for _ in range(2)],
            pltpu.SemaphoreType.DMA,
            [pltpu.SemaphoreType.DMA for _ in range(2)],
        ],
    )
    def k(eo_hbm, d1_hbm, d2_hbm, g1_hbm, g2_hbm, y_hbm,
          r1s, r2s, y_v, i1s, i2s, g1_v, g2_v, sGa, sGb, sI, sY):
        wid = lax.axis_index("s") * _NC + lax.axis_index("c")
        base0 = wid * tpw
        ih = [pltpu.async_copy(g1_hbm.at[pl.ds(base0, tpw)], g1_v, sI),
              pltpu.async_copy(g2_hbm.at[pl.ds(base0, tpw)], g2_v, sI)]
        for c in range(NCH):
            ih.append(pltpu.async_copy(d1_hbm.at[pl.ds(base0 + c * CH, CH)], i1s[c], sI))
            ih.append(pltpu.async_copy(d2_hbm.at[pl.ds(base0 + c * CH, CH)], i2s[c], sI))
        for h in ih:
            h.wait()

        def fire(c):
            par = c % 2
            return (pltpu.async_copy(eo_hbm.at[i1s[c]], r1s[par], sGa[par]),
                    pltpu.async_copy(eo_hbm.at[i2s[c]], r2s[par], sGb[par]))

        gath = [None] * NCH
        gath[0] = fire(0)
        store = [None] * NCH
        for c in range(NCH):
            par = c % 2
            gath[c][0].wait()
            gath[c][1].wait()
            if c + 1 < NCH:
                gath[c + 1] = fire(c + 1)
            if c >= 1:
                store[c - 1].wait()
            r1_v, r2_v = r1s[par], r2s[par]
            for t in range(CH):
                g1b = g1_v[c * CH + t]
                g2b = g2_v[c * CH + t]

                def jbody(j, cc):
                    sl = pl.ds(j * 16, 16)
                    y_v[t, sl] = g1b * r1_v[t, sl] + g2b * r2_v[t, sl]
                    return cc

                lax.fori_loop(0, _C // 16, jbody, 0)
            store[c] = pltpu.async_copy(
                y_v, y_hbm.at[pl.ds(base0 + c * CH, CH)], sY[par])
        store[-1].wait()

    return k(eo, d1, d2, g1, g2)


def kernel(x, Wr1, br1, Wr2, br2, Wr3, br3, We1, be1, We2, be2):
    xt = x.reshape(_N, _C)
    logits = _router_logits(xt, Wr1, br1, Wr2, br2, Wr3, br3)
    d1, d2, g1, g2 = _routing(logits)
    tok_u = lax.bitcast_convert_type(
        xt.astype(jnp.bfloat16).reshape(_N, _C // 2, 2), jnp.int32)
    buf_u = _dispatch(tok_u, d1, d2)
    buf = lax.bitcast_convert_type(buf_u, jnp.bfloat16).reshape(_NROWS, _C)
    eo = _ffn(buf, We1, be1, We2, be2)
    y = _combine(eo, d1, d2, g1, g2)
    return y.reshape(_B, _T, _C)


# speedup vs baseline: 1.4170x; 1.4170x over previous
import functools

import jax
import jax.numpy as jnp
from jax import lax
from jax.experimental import pallas as pl
from jax.experimental.pallas import tpu as pltpu
from jax.experimental.pallas import tpu_sc as plsc

_B, _T, _C = 2, 2048, 1024
_E = 8
_K = 2
_DFF = 4 * _C
_N = _B * _T
_CAP = int(_T / _E * 1.25) * _B
_CPAD = _CAP + 8
_NROWS = _E * _CPAD

_NC, _NS = 2, 16
_NW = _NC * _NS


def _a1_kernel(x_ref, w_ref, b_ref, o_ref):
    acc = jnp.dot(x_ref[...], w_ref[...], preferred_element_type=jnp.float32)
    o_ref[...] = jnp.maximum(acc + b_ref[...], 0.0).astype(jnp.bfloat16)


def _router_h1(xt, Wr1, br1):
    BM = 512
    return pl.pallas_call(
        _a1_kernel,
        grid=(_N // BM,),
        in_specs=[
            pl.BlockSpec((BM, _C), lambda m: (m, 0)),
            pl.BlockSpec((_C, _DFF), lambda m: (0, 0)),
            pl.BlockSpec((1, _DFF), lambda m: (0, 0)),
        ],
        out_specs=pl.BlockSpec((BM, _DFF), lambda m: (m, 0)),
        out_shape=jax.ShapeDtypeStruct((_N, _DFF), jnp.bfloat16),
    )(xt, Wr1, br1.reshape(1, _DFF))


def _a2_kernel(h1_ref, w2_ref, b2_ref, w3_ref, b3_ref, o_ref):
    n = pl.program_id(1)
    h2 = jnp.dot(h1_ref[...], w2_ref[...], preferred_element_type=jnp.float32)
    h2 = jnp.maximum(h2 + b2_ref[...], 0.0)
    part = jnp.dot(h2, w3_ref[...], preferred_element_type=jnp.float32)

    @pl.when(n == 0)
    def _():
        o_ref[...] = part + b3_ref[...]

    @pl.when(n != 0)
    def _():
        o_ref[...] += part


def _router_logits(h1, Wr2, br2, Wr3, br3):
    BM, BN = 1024, 512
    return pl.pallas_call(
        _a2_kernel,
        grid=(_N // BM, _DFF // BN),
        in_specs=[
            pl.BlockSpec((BM, _DFF), lambda m, n: (m, 0)),
            pl.BlockSpec((_DFF, BN), lambda m, n: (0, n)),
            pl.BlockSpec((1, BN), lambda m, n: (0, n)),
            pl.BlockSpec((BN, _E), lambda m, n: (n, 0)),
            pl.BlockSpec((1, _E), lambda m, n: (0, 0)),
        ],
        out_specs=pl.BlockSpec((BM, _E), lambda m, n: (m, 0)),
        out_shape=jax.ShapeDtypeStruct((_N, _E), jnp.float32),
    )(h1, Wr2, br2.reshape(1, _DFF), Wr3, br3.reshape(1, _E))


def _routing_kernel(lg_ref, d1_ref, d2_ref, g1_ref, g2_ref):
    lg = lg_ref[...]
    m = jnp.max(lg, axis=1, keepdims=True)
    p = jnp.exp(lg - m)
    probs = p / jnp.sum(p, axis=1, keepdims=True)
    eidx = lax.broadcasted_iota(jnp.int32, (_N, _E), 1)

    p1 = jnp.max(probs, axis=1, keepdims=True)
    e1 = jnp.min(jnp.where(probs == p1, eidx, _E), axis=1, keepdims=True)
    oh1 = eidx == e1
    probs_m = jnp.where(oh1, -1.0, probs)
    p2 = jnp.max(probs_m, axis=1, keepdims=True)
    e2 = jnp.min(jnp.where(probs_m == p2, eidx, _E), axis=1, keepdims=True)
    oh2 = eidx == e2

    cnt = oh1.astype(jnp.float32) + oh2.astype(jnp.float32)
    c = cnt
    sh = 1
    while sh < _N:
        c = c + jnp.concatenate(
            [jnp.zeros((sh, _E), jnp.float32), c[: _N - sh, :]], axis=0)
        sh *= 2
    cex = c - cnt

    pos1 = jnp.sum(jnp.where(oh1, cex, 0.0), axis=1, keepdims=True)
    pos2 = jnp.sum(jnp.where(oh2, cex, 0.0), axis=1, keepdims=True)
    g1 = jnp.sum(jnp.where(oh1, probs, 0.0), axis=1, keepdims=True)
    g2 = jnp.sum(jnp.where(oh2, probs, 0.0), axis=1, keepdims=True)

    keep1 = pos1 < float(_CAP)
    keep2 = pos2 < float(_CAP)
    p1i = pos1.astype(jnp.int32)
    p2i = pos2.astype(jnp.int32)
    d1 = e1 * _CPAD + jnp.where(keep1, p1i, _CAP)
    d2 = e2 * _CPAD + jnp.where(keep2, p2i, _CAP)

    d1_ref[...] = d1.reshape(_N)
    d2_ref[...] = d2.reshape(_N)
    g1_ref[...] = jnp.broadcast_to(jnp.where(keep1, g1, 0.0), (_N, 16))
    g2_ref[...] = jnp.broadcast_to(jnp.where(keep2, g2, 0.0), (_N, 16))


def _routing(logits):
    return pl.pallas_call(
        _routing_kernel,
        out_shape=(
            jax.ShapeDtypeStruct((_N,), jnp.int32),
            jax.ShapeDtypeStruct((_N,), jnp.int32),
            jax.ShapeDtypeStruct((_N, 16), jnp.float32),
            jax.ShapeDtypeStruct((_N, 16), jnp.float32),
        ),
    )(logits)


def _dispatch(tokens, d1, d2):
    mesh = plsc.VectorSubcoreMesh(core_axis_name="c", subcore_axis_name="s")
    CH = 32
    tpw = _N // _NW
    NCH = tpw // CH

    @functools.partial(
        pl.kernel,
        out_type=jax.ShapeDtypeStruct((_NROWS, _C), jnp.float32),
        mesh=mesh,
        scratch_types=[
            pltpu.VMEM((CH, _C), jnp.float32),
            pltpu.VMEM((CH, _C), jnp.float32),
            [pltpu.VMEM((CH,), jnp.int32) for _ in range(NCH)],
            [pltpu.VMEM((CH,), jnp.int32) for _ in range(NCH)],
            pltpu.SemaphoreType.DMA,
            pltpu.SemaphoreType.DMA,
            pltpu.SemaphoreType.DMA,
            pltpu.SemaphoreType.DMA,
        ],
    )
    def k(tok_hbm, d1_hbm, d2_hbm, buf_hbm, rA, rB, i1s, i2s, sL0, sL1, sS, sI):
        wid = lax.axis_index("s") * _NC + lax.axis_index("c")
        base0 = wid * tpw
        rows = [rA, rB]
        sL = [sL0, sL1]
        ih = []
        for c in range(NCH):
            ih.append(pltpu.async_copy(d1_hbm.at[pl.ds(base0 + c * CH, CH)], i1s[c], sI))
            ih.append(pltpu.async_copy(d2_hbm.at[pl.ds(base0 + c * CH, CH)], i2s[c], sI))
        for h in ih:
            h.wait()
        load = [None] * NCH
        load[0] = pltpu.async_copy(tok_hbm.at[pl.ds(base0, CH)], rows[0], sL[0])
        scat = []
        for c in range(NCH):
            load[c].wait()
            if c >= 1:
                scat[2 * (c - 1)].wait()
                scat[2 * (c - 1) + 1].wait()
            if c + 1 < NCH:
                load[c + 1] = pltpu.async_copy(
                    tok_hbm.at[pl.ds(base0 + (c + 1) * CH, CH)],
                    rows[(c + 1) % 2], sL[(c + 1) % 2])
            scat.append(pltpu.async_copy(rows[c % 2], buf_hbm.at[i1s[c]], sS))
            scat.append(pltpu.async_copy(rows[c % 2], buf_hbm.at[i2s[c]], sS))
        scat[-2].wait()
        scat[-1].wait()

    return k(tokens, d1, d2)


_FFN_BN = 2048


def _ffn_kernel(buf_ref, w1_ref, b1_ref, w2_ref, b2_ref, o_ref):
    n = pl.program_id(1)
    h = jnp.dot(buf_ref[...], w1_ref[0], preferred_element_type=jnp.float32)
    b1 = b1_ref[0, :, pl.ds(n * _FFN_BN, _FFN_BN)]
    h = jnp.maximum(h + b1, 0.0)
    part = jnp.dot(h, w2_ref[0], preferred_element_type=jnp.float32)

    @pl.when(n == 0)
    def _():
        o_ref[...] = part + b2_ref[0]

    @pl.when(n != 0)
    def _():
        o_ref[...] += part


def _ffn(buf, We1, be1, We2, be2):
    BN = _FFN_BN
    return pl.pallas_call(
        _ffn_kernel,
        grid=(_E, _DFF // BN),
        in_specs=[
            pl.BlockSpec((_CPAD, _C), lambda e, n: (e, 0)),
            pl.BlockSpec((1, _C, BN), lambda e, n: (e, 0, n)),
            pl.BlockSpec((1, 1, _DFF), lambda e, n: (e, 0, 0)),
            pl.BlockSpec((1, BN, _C), lambda e, n: (e, n, 0)),
            pl.BlockSpec((1, 1, _C), lambda e, n: (e, 0, 0)),
        ],
        out_specs=pl.BlockSpec((_CPAD, _C), lambda e, n: (e, 0)),
        out_shape=jax.ShapeDtypeStruct((_NROWS, _C), jnp.float32),
    )(buf, We1.reshape(_E, _C, _DFF), be1.reshape(_E, 1, _DFF),
      We2.reshape(_E, _DFF, _C), be2.reshape(_E, 1, _C))


def _combine(eo, d1, d2, g1, g2):
    mesh = plsc.VectorSubcoreMesh(core_axis_name="c", subcore_axis_name="s")
    CH = 16
    tpw = _N // _NW
    NCH = tpw // CH

    @functools.partial(
        pl.kernel,
        out_type=jax.ShapeDtypeStruct((_N, _C), jnp.float32),
        mesh=mesh,
        scratch_types=[
            [pltpu.VMEM((CH, _C), jnp.float32) for _ in range(2)],
            [pltpu.VMEM((CH, _C), jnp.float32) for _ in range(2)],
            pltpu.VMEM((CH, _C), jnp.float32),
            [pltpu.VMEM((CH,), jnp.int32) for _ in range(NCH)],
            [pltpu.VMEM((CH,), jnp.int32) for _ in range(NCH)],
            pltpu.VMEM((tpw, 16), jnp.float32),
            pltpu.VMEM((tpw, 16), jnp.float32),
            [pltpu.SemaphoreType.DMA for _ in range(2)],
            [pltpu.SemaphoreType.DMA for _ in range(2)],
            pltpu.SemaphoreType.DMA,
            [pltpu.SemaphoreType.DMA for _ in range(2)],
        ],
    )
    def k(eo_hbm, d1_hbm, d2_hbm, g1_hbm, g2_hbm, y_hbm,
          r1s, r2s, y_v, i1s, i2s, g1_v, g2_v, sGa, sGb, sI, sY):
        wid = lax.axis_index("s") * _NC + lax.axis_index("c")
        base0 = wid * tpw
        ih = [pltpu.async_copy(g1_hbm.at[pl.ds(base0, tpw)], g1_v, sI),
              pltpu.async_copy(g2_hbm.at[pl.ds(base0, tpw)], g2_v, sI)]
        for c in range(NCH):
            ih.append(pltpu.async_copy(d1_hbm.at[pl.ds(base0 + c * CH, CH)], i1s[c], sI))
            ih.append(pltpu.async_copy(d2_hbm.at[pl.ds(base0 + c * CH, CH)], i2s[c], sI))
        for h in ih:
            h.wait()

        def fire(c):
            par = c % 2
            return (pltpu.async_copy(eo_hbm.at[i1s[c]], r1s[par], sGa[par]),
                    pltpu.async_copy(eo_hbm.at[i2s[c]], r2s[par], sGb[par]))

        gath = [None] * NCH
        gath[0] = fire(0)
        store = [None] * NCH
        for c in range(NCH):
            par = c % 2
            gath[c][0].wait()
            gath[c][1].wait()
            if c + 1 < NCH:
                gath[c + 1] = fire(c + 1)
            if c >= 1:
                store[c - 1].wait()
            r1_v, r2_v = r1s[par], r2s[par]
            for t in range(CH):
                g1b = g1_v[c * CH + t]
                g2b = g2_v[c * CH + t]

                def jbody(j, cc):
                    sl = pl.ds(j * 16, 16)
                    y_v[t, sl] = g1b * r1_v[t, sl] + g2b * r2_v[t, sl]
                    return cc

                lax.fori_loop(0, _C // 16, jbody, 0)
            store[c] = pltpu.async_copy(
                y_v, y_hbm.at[pl.ds(base0 + c * CH, CH)], sY[par])
        store[-1].wait()

    return k(eo, d1, d2, g1, g2)


def kernel(x, Wr1, br1, Wr2, br2, Wr3, br3, We1, be1, We2, be2):
    xt = x.reshape(_N, _C)
    h1 = _router_h1(xt, Wr1, br1)
    logits = _router_logits(h1, Wr2, br2, Wr3, br3)
    d1, d2, g1, g2 = _routing(logits)
    buf = _dispatch(xt, d1, d2)
    eo = _ffn(buf, We1, be1, We2, be2)
    y = _combine(eo, d1, d2, g1, g2)
    return y.reshape(_B, _T, _C)


# speedup vs baseline: 1.4295x; 1.0088x over previous
import functools

import jax
import jax.numpy as jnp
from jax import lax
from jax.experimental import pallas as pl
from jax.experimental.pallas import tpu as pltpu
from jax.experimental.pallas import tpu_sc as plsc

_B, _T, _C = 2, 2048, 1024
_E = 8
_K = 2
_DFF = 4 * _C
_N = _B * _T
_CAP = int(_T / _E * 1.25) * _B
_CPAD = _CAP + 8
_NROWS = _E * _CPAD

_NC, _NS = 2, 16
_NW = _NC * _NS


def _a1_kernel(x_ref, w_ref, b_ref, o_ref):
    acc = jnp.dot(x_ref[...], w_ref[...], preferred_element_type=jnp.float32)
    o_ref[...] = jnp.maximum(acc + b_ref[...], 0.0).astype(jnp.bfloat16)


def _router_h1(xt, Wr1, br1):
    BM = 512
    return pl.pallas_call(
        _a1_kernel,
        grid=(_N // BM,),
        in_specs=[
            pl.BlockSpec((BM, _C), lambda m: (m, 0)),
            pl.BlockSpec((_C, _DFF), lambda m: (0, 0)),
            pl.BlockSpec((1, _DFF), lambda m: (0, 0)),
        ],
        out_specs=pl.BlockSpec((BM, _DFF), lambda m: (m, 0)),
        out_shape=jax.ShapeDtypeStruct((_N, _DFF), jnp.bfloat16),
    )(xt, Wr1, br1.reshape(1, _DFF))


def _a2_kernel(h1_ref, w2_ref, b2_ref, w3_ref, b3_ref, o_ref):
    n = pl.program_id(1)
    h2 = jnp.dot(h1_ref[...], w2_ref[...], preferred_element_type=jnp.float32)
    h2 = jnp.maximum(h2 + b2_ref[...], 0.0)
    part = jnp.dot(h2, w3_ref[...], preferred_element_type=jnp.float32)

    @pl.when(n == 0)
    def _():
        o_ref[...] = part + b3_ref[...]

    @pl.when(n != 0)
    def _():
        o_ref[...] += part


def _router_logits(h1, Wr2, br2, Wr3, br3):
    BM, BN = 2048, 512
    return pl.pallas_call(
        _a2_kernel,
        grid=(_N // BM, _DFF // BN),
        in_specs=[
            pl.BlockSpec((BM, _DFF), lambda m, n: (m, 0)),
            pl.BlockSpec((_DFF, BN), lambda m, n: (0, n)),
            pl.BlockSpec((1, BN), lambda m, n: (0, n)),
            pl.BlockSpec((BN, _E), lambda m, n: (n, 0)),
            pl.BlockSpec((1, _E), lambda m, n: (0, 0)),
        ],
        out_specs=pl.BlockSpec((BM, _E), lambda m, n: (m, 0)),
        out_shape=jax.ShapeDtypeStruct((_N, _E), jnp.float32),
    )(h1, Wr2, br2.reshape(1, _DFF), Wr3, br3.reshape(1, _E))


def _routing_kernel(lg_ref, d1_ref, d2_ref, g1_ref, g2_ref):
    lg = lg_ref[...]
    m = jnp.max(lg, axis=1, keepdims=True)
    p = jnp.exp(lg - m)
    probs = p / jnp.sum(p, axis=1, keepdims=True)
    eidx = lax.broadcasted_iota(jnp.int32, (_N, _E), 1)

    p1 = jnp.max(probs, axis=1, keepdims=True)
    e1 = jnp.min(jnp.where(probs == p1, eidx, _E), axis=1, keepdims=True)
    oh1 = eidx == e1
    probs_m = jnp.where(oh1, -1.0, probs)
    p2 = jnp.max(probs_m, axis=1, keepdims=True)
    e2 = jnp.min(jnp.where(probs_m == p2, eidx, _E), axis=1, keepdims=True)
    oh2 = eidx == e2

    cnt = oh1.astype(jnp.float32) + oh2.astype(jnp.float32)
    c = cnt
    sh = 1
    while sh < _N:
        c = c + jnp.concatenate(
            [jnp.zeros((sh, _E), jnp.float32), c[: _N - sh, :]], axis=0)
        sh *= 2
    cex = c - cnt

    pos1 = jnp.sum(jnp.where(oh1, cex, 0.0), axis=1, keepdims=True)
    pos2 = jnp.sum(jnp.where(oh2, cex, 0.0), axis=1, keepdims=True)
    g1 = jnp.sum(jnp.where(oh1, probs, 0.0), axis=1, keepdims=True)
    g2 = jnp.sum(jnp.where(oh2, probs, 0.0), axis=1, keepdims=True)

    keep1 = pos1 < float(_CAP)
    keep2 = pos2 < float(_CAP)
    p1i = pos1.astype(jnp.int32)
    p2i = pos2.astype(jnp.int32)
    d1 = e1 * _CPAD + jnp.where(keep1, p1i, _CAP)
    d2 = e2 * _CPAD + jnp.where(keep2, p2i, _CAP)

    d1_ref[...] = d1.reshape(_N)
    d2_ref[...] = d2.reshape(_N)
    g1_ref[...] = jnp.broadcast_to(jnp.where(keep1, g1, 0.0), (_N, 16))
    g2_ref[...] = jnp.broadcast_to(jnp.where(keep2, g2, 0.0), (_N, 16))


def _routing(logits):
    return pl.pallas_call(
        _routing_kernel,
        out_shape=(
            jax.ShapeDtypeStruct((_N,), jnp.int32),
            jax.ShapeDtypeStruct((_N,), jnp.int32),
            jax.ShapeDtypeStruct((_N, 16), jnp.float32),
            jax.ShapeDtypeStruct((_N, 16), jnp.float32),
        ),
    )(logits)


def _dispatch(tokens, d1, d2):
    mesh = plsc.VectorSubcoreMesh(core_axis_name="c", subcore_axis_name="s")
    CH = 32
    tpw = _N // _NW
    NCH = tpw // CH

    @functools.partial(
        pl.kernel,
        out_type=jax.ShapeDtypeStruct((_NROWS, _C), jnp.float32),
        mesh=mesh,
        scratch_types=[
            pltpu.VMEM((CH, _C), jnp.float32),
            pltpu.VMEM((CH, _C), jnp.float32),
            [pltpu.VMEM((CH,), jnp.int32) for _ in range(NCH)],
            [pltpu.VMEM((CH,), jnp.int32) for _ in range(NCH)],
            pltpu.SemaphoreType.DMA,
            pltpu.SemaphoreType.DMA,
            pltpu.SemaphoreType.DMA,
            pltpu.SemaphoreType.DMA,
        ],
    )
    def k(tok_hbm, d1_hbm, d2_hbm, buf_hbm, rA, rB, i1s, i2s, sL0, sL1, sS, sI):
        wid = lax.axis_index("s") * _NC + lax.axis_index("c")
        base0 = wid * tpw
        rows = [rA, rB]
        sL = [sL0, sL1]
        ih = []
        for c in range(NCH):
            ih.append(pltpu.async_copy(d1_hbm.at[pl.ds(base0 + c * CH, CH)], i1s[c], sI))
            ih.append(pltpu.async_copy(d2_hbm.at[pl.ds(base0 + c * CH, CH)], i2s[c], sI))
        for h in ih:
            h.wait()
        load = [None] * NCH
        load[0] = pltpu.async_copy(tok_hbm.at[pl.ds(base0, CH)], rows[0], sL[0])
        scat = []
        for c in range(NCH):
            load[c].wait()
            if c >= 1:
                scat[2 * (c - 1)].wait()
                scat[2 * (c - 1) + 1].wait()
            if c + 1 < NCH:
                load[c + 1] = pltpu.async_copy(
                    tok_hbm.at[pl.ds(base0 + (c + 1) * CH, CH)],
                    rows[(c + 1) % 2], sL[(c + 1) % 2])
            scat.append(pltpu.async_copy(rows[c % 2], buf_hbm.at[i1s[c]], sS))
            scat.append(pltpu.async_copy(rows[c % 2], buf_hbm.at[i2s[c]], sS))
        scat[-2].wait()
        scat[-1].wait()

    return k(tokens, d1, d2)


_FFN_BN = 2048


def _ffn_kernel(buf_ref, w1_ref, b1_ref, w2_ref, b2_ref, o_ref):
    n = pl.program_id(1)
    h = jnp.dot(buf_ref[...], w1_ref[0], preferred_element_type=jnp.float32)
    b1 = b1_ref[0, :, pl.ds(n * _FFN_BN, _FFN_BN)]
    h = jnp.maximum(h + b1, 0.0)
    part = jnp.dot(h, w2_ref[0], preferred_element_type=jnp.float32)

    @pl.when(n == 0)
    def _():
        o_ref[...] = part + b2_ref[0]

    @pl.when(n != 0)
    def _():
        o_ref[...] += part


def _ffn(buf, We1, be1, We2, be2):
    BN = _FFN_BN
    return pl.pallas_call(
        _ffn_kernel,
        grid=(_E, _DFF // BN),
        in_specs=[
            pl.BlockSpec((_CPAD, _C), lambda e, n: (e, 0)),
            pl.BlockSpec((1, _C, BN), lambda e, n: (e, 0, n)),
            pl.BlockSpec((1, 1, _DFF), lambda e, n: (e, 0, 0)),
            pl.BlockSpec((1, BN, _C), lambda e, n: (e, n, 0)),
            pl.BlockSpec((1, 1, _C), lambda e, n: (e, 0, 0)),
        ],
        out_specs=pl.BlockSpec((_CPAD, _C), lambda e, n: (e, 0)),
        out_shape=jax.ShapeDtypeStruct((_NROWS, _C), jnp.float32),
    )(buf, We1.reshape(_E, _C, _DFF), be1.reshape(_E, 1, _DFF),
      We2.reshape(_E, _DFF, _C), be2.reshape(_E, 1, _C))


def _combine(eo, d1, d2, g1, g2):
    mesh = plsc.VectorSubcoreMesh(core_axis_name="c", subcore_axis_name="s")
    CH = 16
    tpw = _N // _NW
    NCH = tpw // CH

    @functools.partial(
        pl.kernel,
        out_type=jax.ShapeDtypeStruct((_N, _C), jnp.float32),
        mesh=mesh,
        scratch_types=[
            [pltpu.VMEM((CH, _C), jnp.float32) for _ in range(2)],
            [pltpu.VMEM((CH, _C), jnp.float32) for _ in range(2)],
            pltpu.VMEM((CH, _C), jnp.float32),
            [pltpu.VMEM((CH,), jnp.int32) for _ in range(NCH)],
            [pltpu.VMEM((CH,), jnp.int32) for _ in range(NCH)],
            pltpu.VMEM((tpw, 16), jnp.float32),
            pltpu.VMEM((tpw, 16), jnp.float32),
            [pltpu.SemaphoreType.DMA for _ in range(2)],
            [pltpu.SemaphoreType.DMA for _ in range(2)],
            pltpu.SemaphoreType.DMA,
            [pltpu.SemaphoreType.DMA for _ in range(2)],
        ],
    )
    def k(eo_hbm, d1_hbm, d2_hbm, g1_hbm, g2_hbm, y_hbm,
          r1s, r2s, y_v, i1s, i2s, g1_v, g2_v, sGa, sGb, sI, sY):
        wid = lax.axis_index("s") * _NC + lax.axis_index("c")
        base0 = wid * tpw
        ih = [pltpu.async_copy(g1_hbm.at[pl.ds(base0, tpw)], g1_v, sI),
              pltpu.async_copy(g2_hbm.at[pl.ds(base0, tpw)], g2_v, sI)]
        for c in range(NCH):
            ih.append(pltpu.async_copy(d1_hbm.at[pl.ds(base0 + c * CH, CH)], i1s[c], sI))
            ih.append(pltpu.async_copy(d2_hbm.at[pl.ds(base0 + c * CH, CH)], i2s[c], sI))
        for h in ih:
            h.wait()

        def fire(c):
            par = c % 2
            return (pltpu.async_copy(eo_hbm.at[i1s[c]], r1s[par], sGa[par]),
                    pltpu.async_copy(eo_hbm.at[i2s[c]], r2s[par], sGb[par]))

        gath = [None] * NCH
        gath[0] = fire(0)
        store = [None] * NCH
        for c in range(NCH):
            par = c % 2
            gath[c][0].wait()
            gath[c][1].wait()
            if c + 1 < NCH:
                gath[c + 1] = fire(c + 1)
            if c >= 1:
                store[c - 1].wait()
            r1_v, r2_v = r1s[par], r2s[par]
            for t in range(CH):
                g1b = g1_v[c * CH + t]
                g2b = g2_v[c * CH + t]

                def jbody(j, cc):
                    sl = pl.ds(j * 16, 16)
                    y_v[t, sl] = g1b * r1_v[t, sl] + g2b * r2_v[t, sl]
                    return cc

                lax.fori_loop(0, _C // 16, jbody, 0)
            store[c] = pltpu.async_copy(
                y_v, y_hbm.at[pl.ds(base0 + c * CH, CH)], sY[par])
        store[-1].wait()

    return k(eo, d1, d2, g1, g2)


def kernel(x, Wr1, br1, Wr2, br2, Wr3, br3, We1, be1, We2, be2):
    xt = x.reshape(_N, _C)
    h1 = _router_h1(xt, Wr1, br1)
    logits = _router_logits(h1, Wr2, br2, Wr3, br3)
    d1, d2, g1, g2 = _routing(logits)
    buf = _dispatch(xt, d1, d2)
    eo = _ffn(buf, We1, be1, We2, be2)
    y = _combine(eo, d1, d2, g1, g2)
    return y.reshape(_B, _T, _C)
